# baseline (speedup 1.0000x reference)
import math

import jax
import jax.numpy as jnp
from jax.experimental import pallas as pl
from jax.experimental.pallas import tpu as pltpu

_KNN = 8
_INV_SP2 = 1.0 / (0.03 ** 2)
_INV_C2 = 1.0 / (1.0 - math.cos(math.radians(15.0)))


def _loss_kernel(preds_ref, gb_ref, rhsn_ref, nrmb_ref, gg_ref, c_ref,
                 out_ref):
    q = preds_ref[0]
    gb = gb_ref[0]
    rhsn = rhsn_ref[0]
    nrmb = nrmb_ref[0]
    gg = gg_ref[0]
    c = c_ref[0]

    m = nrmb.shape[0]
    f32 = jnp.float32
    bf16 = jnp.bfloat16

    qq = jnp.sum(q * q, axis=1, keepdims=True)

    q_hi = q.astype(bf16)
    q_lo = (q - q_hi.astype(f32)).astype(bf16)
    lhs = jnp.concatenate([q_hi, q_lo, q_hi], axis=1)

    dims_t = (((1,), (1,)), ((), ()))
    qg = jax.lax.dot_general(q_hi, gb, dims_t,
                             preferred_element_type=f32)
    p = jax.lax.dot_general(lhs, rhsn, dims_t,
                            preferred_element_type=f32)

    d = (qq + gg) - 2.0 * qg
    ip = jnp.abs(p - c)

    big = jnp.float32(jnp.inf)

    t = jnp.min(d, axis=1, keepdims=True)
    eq = (d == t).astype(bf16)
    e = jax.lax.dot_general(eq, nrmb, (((1,), (0,)), ((), ())),
                            preferred_element_type=f32)

    for _ in range(_KNN - 1):
        t = jnp.min(jnp.where(d > t, d, big), axis=1, keepdims=True)

    inner_n = jax.lax.dot_general(e.astype(bf16), nrmb, dims_t,
                                  preferred_element_type=f32)

    w = jnp.where(d <= t, jnp.exp(d * (-_INV_SP2) + (inner_n - 1.0) * _INV_C2),
                  0.0)

    num = jnp.sum(w * ip, axis=1, keepdims=True)
    den = jnp.sum(w, axis=1, keepdims=True)
    tile_sum = jnp.sum(num / den).reshape(1, 1, 1)

    @pl.when(jnp.logical_and(pl.program_id(0) == 0, pl.program_id(1) == 0))
    def _init():
        out_ref[:, :, :] = jnp.zeros((1, 1, 1), jnp.float32)

    out_ref[:, :, :] += tile_sum


def kernel(preds, gts, normals):
    b, n, _ = preds.shape
    m = gts.shape[1]
    nq = 256
    f32 = jnp.float32
    bf16 = jnp.bfloat16

    n_hi = normals.astype(bf16)
    n_lo = (normals - n_hi.astype(f32)).astype(bf16)
    rhsn = jnp.concatenate([n_hi, n_hi, n_lo], axis=2)

    gg = jnp.sum(gts * gts, axis=-1)[:, None, :]
    c = jnp.sum(gts * normals, axis=-1)[:, None, :]

    out = pl.pallas_call(
        _loss_kernel,
        grid=(b, n // nq),
        in_specs=[
            pl.BlockSpec((1, nq, 3), lambda bi, i: (bi, i, 0)),
            pl.BlockSpec((1, m, 3), lambda bi, i: (bi, 0, 0)),
            pl.BlockSpec((1, m, 9), lambda bi, i: (bi, 0, 0)),
            pl.BlockSpec((1, m, 3), lambda bi, i: (bi, 0, 0)),
            pl.BlockSpec((1, 1, m), lambda bi, i: (bi, 0, 0)),
            pl.BlockSpec((1, 1, m), lambda bi, i: (bi, 0, 0)),
        ],
        out_specs=pl.BlockSpec((1, 1, 1), lambda bi, i: (0, 0, 0)),
        out_shape=jax.ShapeDtypeStruct((1, 1, 1), jnp.float32),
    )(preds, gts.astype(bf16), rhsn, normals.astype(bf16), gg, c)
    return out[0, 0, 0]


# speedup vs baseline: 1.0109x; 1.0109x over previous
import math

import jax
import jax.numpy as jnp
from jax.experimental import pallas as pl

_KNN = 8
_INV_SP2 = 1.0 / (0.03 ** 2)
_INV_C2 = 1.0 / (1.0 - math.cos(math.radians(15.0)))


def _loss_kernel(preds_ref, rhs_ref, nrmb_ref, gg_ref, c_ref, out_ref):
    q = preds_ref[0]
    rhs = rhs_ref[0]
    nrmb = nrmb_ref[0]
    gg = gg_ref[0]
    c = c_ref[0]

    m = nrmb.shape[0]
    f32 = jnp.float32
    bf16 = jnp.bfloat16

    qq = jnp.sum(q * q, axis=1, keepdims=True)

    q_hi = q.astype(bf16)
    q_lo = (q - q_hi.astype(f32)).astype(bf16)
    lhs = jnp.concatenate([q_hi, q_lo, q_hi], axis=1)

    dims_t = (((1,), (1,)), ((), ()))
    both = jax.lax.dot_general(lhs, rhs, dims_t,
                               preferred_element_type=f32)
    qg = both[:, :m]
    p = both[:, m:]

    d = (qq + gg) - 2.0 * qg
    ip = jnp.abs(p - c)

    big = jnp.float32(jnp.inf)

    t = jnp.min(d, axis=1, keepdims=True)
    eq = (d == t).astype(bf16)
    e = jax.lax.dot_general(eq, nrmb, (((1,), (0,)), ((), ())),
                            preferred_element_type=f32)

    for _ in range(_KNN - 1):
        t = jnp.min(jnp.where(d > t, d, big), axis=1, keepdims=True)

    inner_n = jax.lax.dot_general(e.astype(bf16), nrmb, dims_t,
                                  preferred_element_type=f32)

    w = jnp.where(d <= t, jnp.exp(d * (-_INV_SP2) + (inner_n - 1.0) * _INV_C2),
                  0.0)

    num = jnp.sum(w * ip, axis=1, keepdims=True)
    den = jnp.sum(w, axis=1, keepdims=True)
    tile_sum = jnp.sum(num / den).reshape(1, 1, 1)

    @pl.when(jnp.logical_and(pl.program_id(0) == 0, pl.program_id(1) == 0))
    def _init():
        out_ref[:, :, :] = jnp.zeros((1, 1, 1), jnp.float32)

    out_ref[:, :, :] += tile_sum


def kernel(preds, gts, normals):
    b, n, _ = preds.shape
    m = gts.shape[1]
    nq = 256
    f32 = jnp.float32
    bf16 = jnp.bfloat16

    n_hi = normals.astype(bf16)
    n_lo = (normals - n_hi.astype(f32)).astype(bf16)
    zeros6 = jnp.zeros((b, m, 6), dtype=bf16)
    rhs = jnp.concatenate([
        jnp.concatenate([gts.astype(bf16), zeros6], axis=2),
        jnp.concatenate([n_hi, n_hi, n_lo], axis=2),
    ], axis=1)

    gg = jnp.sum(gts * gts, axis=-1)[:, None, :]
    c = jnp.sum(gts * normals, axis=-1)[:, None, :]

    out = pl.pallas_call(
        _loss_kernel,
        grid=(b, n // nq),
        in_specs=[
            pl.BlockSpec((1, nq, 3), lambda bi, i: (bi, i, 0)),
            pl.BlockSpec((1, 2 * m, 9), lambda bi, i: (bi, 0, 0)),
            pl.BlockSpec((1, m, 3), lambda bi, i: (bi, 0, 0)),
            pl.BlockSpec((1, 1, m), lambda bi, i: (bi, 0, 0)),
            pl.BlockSpec((1, 1, m), lambda bi, i: (bi, 0, 0)),
        ],
        out_specs=pl.BlockSpec((1, 1, 1), lambda bi, i: (0, 0, 0)),
        out_shape=jax.ShapeDtypeStruct((1, 1, 1), jnp.float32),
    )(preds, rhs, normals.astype(bf16), gg, c)
    return out[0, 0, 0]


# speedup vs baseline: 1.0316x; 1.0205x over previous
import math

import jax
import jax.numpy as jnp
from jax.experimental import pallas as pl

_KNN = 8
_INV_SP2 = 1.0 / (0.03 ** 2)
_INV_C2 = 1.0 / (1.0 - math.cos(math.radians(15.0)))
_LOG2E = math.log2(math.e)


def _loss_kernel(preds_ref, rhs_ref, nrmb_ref, gg_ref, c_ref, out_ref):
    q = preds_ref[0]
    rhs = rhs_ref[0]
    nrmb = nrmb_ref[0]
    gg = gg_ref[0]
    c = c_ref[0]

    m = nrmb.shape[0]
    f32 = jnp.float32
    bf16 = jnp.bfloat16

    qq = jnp.sum(q * q, axis=1, keepdims=True)

    q_hi = q.astype(bf16)
    q_lo = (q - q_hi.astype(f32)).astype(bf16)
    lhs = jnp.concatenate([q_hi, q_lo, q_hi], axis=1)

    dims_t = (((1,), (1,)), ((), ()))
    both = jax.lax.dot_general(lhs, rhs, dims_t,
                               preferred_element_type=f32)
    qg = both[:, :m]
    p = both[:, m:]

    d = (qq + gg) - 2.0 * qg
    ip = jnp.abs(p - c)

    big = jnp.float32(jnp.inf)

    t = jnp.min(d, axis=1, keepdims=True)
    eq = (d == t).astype(bf16)
    e = jax.lax.dot_general(eq, nrmb, (((1,), (0,)), ((), ())),
                            preferred_element_type=f32)

    for _ in range(_KNN - 1):
        t = jnp.min(jnp.where(d > t, d, big), axis=1, keepdims=True)

    inner_n = jax.lax.dot_general(e.astype(bf16), nrmb, dims_t,
                                  preferred_element_type=f32)

    w = jnp.where(d <= t,
                  jnp.exp2(d * (-_INV_SP2 * _LOG2E)
                           + (inner_n - 1.0) * (_INV_C2 * _LOG2E)), 0.0)

    num = jnp.sum(w * ip, axis=1, keepdims=True)
    den = jnp.sum(w, axis=1, keepdims=True)
    tile_sum = jnp.sum(num / den).reshape(1, 1, 1)

    @pl.when(jnp.logical_and(pl.program_id(0) == 0, pl.program_id(1) == 0))
    def _init():
        out_ref[:, :, :] = jnp.zeros((1, 1, 1), jnp.float32)

    out_ref[:, :, :] += tile_sum


def kernel(preds, gts, normals):
    b, n, _ = preds.shape
    m = gts.shape[1]
    nq = 256
    f32 = jnp.float32
    bf16 = jnp.bfloat16

    n_hi = normals.astype(bf16)
    n_lo = (normals - n_hi.astype(f32)).astype(bf16)
    zeros6 = jnp.zeros((b, m, 6), dtype=bf16)
    rhs = jnp.concatenate([
        jnp.concatenate([gts.astype(bf16), zeros6], axis=2),
        jnp.concatenate([n_hi, n_hi, n_lo], axis=2),
    ], axis=1)

    gg = jnp.sum(gts * gts, axis=-1)[:, None, :]
    c = jnp.sum(gts * normals, axis=-1)[:, None, :]

    out = pl.pallas_call(
        _loss_kernel,
        grid=(b, n // nq),
        in_specs=[
            pl.BlockSpec((1, nq, 3), lambda bi, i: (bi, i, 0)),
            pl.BlockSpec((1, 2 * m, 9), lambda bi, i: (bi, 0, 0)),
            pl.BlockSpec((1, m, 3), lambda bi, i: (bi, 0, 0)),
            pl.BlockSpec((1, 1, m), lambda bi, i: (bi, 0, 0)),
            pl.BlockSpec((1, 1, m), lambda bi, i: (bi, 0, 0)),
        ],
        out_specs=pl.BlockSpec((1, 1, 1), lambda bi, i: (0, 0, 0)),
        out_shape=jax.ShapeDtypeStruct((1, 1, 1), jnp.float32),
    )(preds, rhs, normals.astype(bf16), gg, c)
    return out[0, 0, 0]
